# baseline (speedup 1.0000x reference)
import functools

import jax
import jax.numpy as jnp
from jax import lax
from jax.experimental import pallas as pl
from jax.experimental.pallas import tpu as pltpu
from jax.experimental.pallas import tpu_sc as plsc

P = 97
OP_ADD = P
OP_SUB = P + 1
OP_MUL = P + 2
PAD_TOK = P + 3 + 3
N = 32768
NT = 32
CHUNK = N // NT
LOC = CHUNK + 16
NV = CHUNK // 16
BIG = 1 << 29


def _phase1(tok_hbm, bv_hbm, comp_hbm, kmat_hbm, cmat_hbm,
            tv, st, pref, comp, bvec, kvec, sem):
  del sem
  wid = lax.axis_index("c") * 16 + lax.axis_index("s")

  if True:
    base = wid * CHUNK

    pltpu.sync_copy(tok_hbm.at[pl.ds(base, LOC)], tv)
    pltpu.sync_copy(bv_hbm, bvec)
    bv = bvec[...]

    def _zero(k, _):
      st[pl.ds(k * 16, 16)] = jnp.zeros((16,), jnp.int32)
      return 0
    lax.fori_loop(0, LOC // 16, _zero, 0)
    kb = bv + 8 - base
    bmask = (kb >= 0) & (kb < LOC)
    plsc.store_scatter(st, [jnp.where(bmask, kb, 0)],
                       jnp.ones((16,), jnp.int32), mask=bmask)

    def _step(k, wp):
      l0 = 8 + k * 16
      t0 = tv[pl.ds(l0, 16)]
      tm1 = tv[pl.ds(l0 - 1, 16)]
      tm2 = tv[pl.ds(l0 - 2, 16)]
      tp1 = tv[pl.ds(l0 + 1, 16)]
      tp2 = tv[pl.ds(l0 + 2, 16)]
      s0 = st[pl.ds(l0, 16)] > 0
      sm1 = st[pl.ds(l0 - 1, 16)] > 0
      sp1 = st[pl.ds(l0 + 1, 16)] > 0
      sp2 = st[pl.ds(l0 + 2, 16)] > 0

      def opnd(x):
        return x < P

      def opr(x):
        return (x >= P) & (x < P + 3)

      red0 = opr(t0) & opnd(tm1) & opnd(tm2) & (~s0) & (~sm1)
      red1 = opr(tp1) & opnd(t0) & opnd(tm1) & (~sp1) & (~s0)
      red2 = opr(tp2) & opnd(tp1) & opnd(t0) & (~sp2) & (~sp1)
      keep = ~(red0 | red1)

      r_add = lax.rem(t0 + tp1, P)
      r_sub = lax.rem(t0 - tp1 + P, P)
      r_mul = lax.rem(t0 * tp1, P)
      res = jnp.where(tp2 == OP_ADD, r_add,
                      jnp.where(tp2 == OP_SUB, r_sub, r_mul))
      tok_new = jnp.where(red2, res, t0)

      ki = keep.astype(jnp.int32)
      pref[pl.ds(k * 16, 16)] = wp + plsc.cumsum(ki) - ki
      plsc.store_compressed(comp.at[pl.ds(wp, 16)], tok_new, mask=keep)
      return wp + jnp.sum(ki)

    k_t = lax.fori_loop(0, NV, _step, jnp.int32(0))

    pltpu.sync_copy(comp.at[pl.ds(0, CHUNK)], comp_hbm.at[pl.ds(base, CHUNK)])
    kvec[...] = jnp.full((16,), k_t, jnp.int32)
    pltpu.sync_copy(kvec, kmat_hbm.at[wid])

    loc = bv - base
    inr = (loc >= 0) & (loc < CHUNK)
    g = plsc.load_gather(pref, [jnp.where(inr, loc, 0)])
    kvec[...] = jnp.where(inr, g, -1)
    pltpu.sync_copy(kvec, cmat_hbm.at[wid])


def _phase2(comp_hbm, kmat_hbm, cmat_hbm, out_tok, out_val, out_cu,
            comp, fcomp, cidx, pidx, padt, padv, kall, call2, cuv, sem):
  wid = lax.axis_index("c") * 16 + lax.axis_index("s")

  if True:
    base = wid * CHUNK
    lane = lax.iota(jnp.int32, 16)

    pltpu.sync_copy(kmat_hbm, kall)
    pltpu.sync_copy(comp_hbm.at[pl.ds(base, CHUNK)], comp)

    def _scan(r, c):
      b, t = c
      kr = kall[r, :][0]
      return (b + jnp.where(r < wid, kr, 0), t + kr)

    base_t, k_tot = lax.fori_loop(0, NT, _scan, (jnp.int32(0), jnp.int32(0)))
    k_t = kall[wid, :][0]

    def _mk(k, _):
      flat = k * 16 + lane
      tok_c = comp[pl.ds(k * 16, 16)]
      fcomp[pl.ds(k * 16, 16)] = tok_c.astype(jnp.float32)
      j = k // 8
      c = k % 8
      cidx[j, pl.ds(c * 16, 16)] = jnp.where(flat < k_t, base_t + flat,
                                             N + flat)
      p = base + flat
      pidx[j, pl.ds(c * 16, 16)] = jnp.where(p >= k_tot, p, N + flat)
      return 0
    lax.fori_loop(0, NV, _mk, 0)

    def _pd(k, _):
      padt[pl.ds(k * 16, 16)] = jnp.full((16,), PAD_TOK, jnp.int32)
      padv[pl.ds(k * 16, 16)] = jnp.zeros((16,), jnp.float32)
      return 0
    lax.fori_loop(0, 8, _pd, 0)

    cps = []
    for j in range(CHUNK // 128):
      cps.append(pltpu.async_copy(comp.at[pl.ds(128 * j, 128)],
                                  out_tok.at[cidx.at[j]], sem))
      cps.append(pltpu.async_copy(fcomp.at[pl.ds(128 * j, 128)],
                                  out_val.at[cidx.at[j]], sem))
      cps.append(pltpu.async_copy(padt, out_tok.at[pidx.at[j]], sem))
      cps.append(pltpu.async_copy(padv, out_val.at[pidx.at[j]], sem))
    for cp in cps:
      cp.wait()

    @pl.when(wid == 0)
    def _cu():
      pltpu.sync_copy(cmat_hbm, call2)

      def _sum(r, c):
        acc, run = c
        row = call2[r, :]
        acc = acc + jnp.where(row >= 0, run + row, 0)
        return (acc, run + kall[r, :][0])

      acc, _ = lax.fori_loop(0, NT, _sum,
                             (jnp.zeros((16,), jnp.int32), jnp.int32(0)))
      cuv[pl.ds(0, 16)] = acc
      cuv[pl.ds(16, 16)] = jnp.where(lane == 0, k_tot, 0)
      pltpu.sync_copy(cuv, out_cu)


_p1 = functools.partial(
    pl.kernel,
    out_type=[
        jax.ShapeDtypeStruct((N,), jnp.int32),
        jax.ShapeDtypeStruct((32, 16), jnp.int32),
        jax.ShapeDtypeStruct((32, 16), jnp.int32),
    ],
    mesh=plsc.VectorSubcoreMesh(core_axis_name="c", subcore_axis_name="s"),
    compiler_params=pltpu.CompilerParams(needs_layout_passes=False),
    scratch_types=[
        pltpu.VMEM((LOC,), jnp.int32),
        pltpu.VMEM((LOC,), jnp.int32),
        pltpu.VMEM((CHUNK,), jnp.int32),
        pltpu.VMEM((LOC,), jnp.int32),
        pltpu.VMEM((16,), jnp.int32),
        pltpu.VMEM((16,), jnp.int32),
        pltpu.SemaphoreType.DMA,
    ],
)(_phase1)


_p2 = functools.partial(
    pl.kernel,
    out_type=[
        jax.ShapeDtypeStruct((N + CHUNK,), jnp.int32),
        jax.ShapeDtypeStruct((N + CHUNK,), jnp.float32),
        jax.ShapeDtypeStruct((32,), jnp.int32),
    ],
    mesh=plsc.VectorSubcoreMesh(core_axis_name="c", subcore_axis_name="s"),
    compiler_params=pltpu.CompilerParams(needs_layout_passes=False),
    scratch_types=[
        pltpu.VMEM((CHUNK,), jnp.int32),
        pltpu.VMEM((CHUNK,), jnp.float32),
        pltpu.VMEM((8, 128), jnp.int32),
        pltpu.VMEM((8, 128), jnp.int32),
        pltpu.VMEM((128,), jnp.int32),
        pltpu.VMEM((128,), jnp.float32),
        pltpu.VMEM((32, 16), jnp.int32),
        pltpu.VMEM((32, 16), jnp.int32),
        pltpu.VMEM((32,), jnp.int32),
        pltpu.SemaphoreType.DMA,
    ],
)(_phase2)


def kernel(tokens, cu_seqlens, values_f):
  del values_f
  pad8 = jnp.full((8,), PAD_TOK, jnp.int32)
  tokens_p = jnp.concatenate([pad8, tokens.astype(jnp.int32), pad8])
  bv = jnp.concatenate(
      [jnp.full((1,), BIG, jnp.int32), cu_seqlens[1:16].astype(jnp.int32)])
  comp_s, kmat, cmat = _p1(tokens_p, bv)
  out_tok, out_val, out_cu = _p2(comp_s, kmat, cmat)
  return out_tok[:N], out_val[:N], out_cu[:17]


# speedup vs baseline: 1.3325x; 1.3325x over previous
import functools

import jax
import jax.numpy as jnp
from jax import lax
from jax.experimental import pallas as pl
from jax.experimental.pallas import tpu as pltpu
from jax.experimental.pallas import tpu_sc as plsc

P = 97
OP_ADD = P
OP_SUB = P + 1
OP_MUL = P + 2
PAD_TOK = P + 3 + 3
N = 32768
NT = 16
CHUNK = N // NT
LOC = CHUNK + 16
NV = CHUNK // 16
BIG = 1 << 29


def _phase1(tok_hbm, bv_hbm, comp_hbm, kmat_hbm, cmat_hbm,
            tv, st, pref, comp, bvec, kvec, sem):
  del sem
  cid = lax.axis_index("c")
  sid = lax.axis_index("s")

  @pl.when(cid == 0)
  def _():
    base = sid * CHUNK

    pltpu.sync_copy(tok_hbm.at[pl.ds(base, LOC)], tv)
    pltpu.sync_copy(bv_hbm, bvec)
    bv = bvec[...]

    def _zero(k, _):
      st[pl.ds(k * 16, 16)] = jnp.zeros((16,), jnp.int32)
      return 0
    lax.fori_loop(0, LOC // 16, _zero, 0)
    kb = bv + 8 - base
    bmask = (kb >= 0) & (kb < LOC)
    plsc.store_scatter(st, [jnp.where(bmask, kb, 0)],
                       jnp.ones((16,), jnp.int32), mask=bmask)

    def _step(k, wp):
      l0 = 8 + k * 16
      t0 = tv[pl.ds(l0, 16)]
      tm1 = tv[pl.ds(l0 - 1, 16)]
      tm2 = tv[pl.ds(l0 - 2, 16)]
      tp1 = tv[pl.ds(l0 + 1, 16)]
      tp2 = tv[pl.ds(l0 + 2, 16)]
      s0 = st[pl.ds(l0, 16)] > 0
      sm1 = st[pl.ds(l0 - 1, 16)] > 0
      sp1 = st[pl.ds(l0 + 1, 16)] > 0
      sp2 = st[pl.ds(l0 + 2, 16)] > 0

      def opnd(x):
        return x < P

      def opr(x):
        return (x >= P) & (x < P + 3)

      red0 = opr(t0) & opnd(tm1) & opnd(tm2) & (~s0) & (~sm1)
      red1 = opr(tp1) & opnd(t0) & opnd(tm1) & (~sp1) & (~s0)
      red2 = opr(tp2) & opnd(tp1) & opnd(t0) & (~sp2) & (~sp1)
      keep = ~(red0 | red1)

      r_add = lax.rem(t0 + tp1, P)
      r_sub = lax.rem(t0 - tp1 + P, P)
      r_mul = lax.rem(t0 * tp1, P)
      res = jnp.where(tp2 == OP_ADD, r_add,
                      jnp.where(tp2 == OP_SUB, r_sub, r_mul))
      tok_new = jnp.where(red2, res, t0)

      ki = keep.astype(jnp.int32)
      pref[pl.ds(k * 16, 16)] = wp + plsc.cumsum(ki) - ki
      plsc.store_compressed(comp.at[pl.ds(wp, 16)], tok_new, mask=keep)
      return wp + jnp.sum(ki)

    k_t = lax.fori_loop(0, NV, _step, jnp.int32(0))

    pltpu.sync_copy(comp.at[pl.ds(0, CHUNK)], comp_hbm.at[pl.ds(base, CHUNK)])
    kvec[...] = jnp.full((16,), k_t, jnp.int32)
    pltpu.sync_copy(kvec, kmat_hbm.at[sid])

    loc = bv - base
    inr = (loc >= 0) & (loc < CHUNK)
    g = plsc.load_gather(pref, [jnp.where(inr, loc, 0)])
    kvec[...] = jnp.where(inr, g, -1)
    pltpu.sync_copy(kvec, cmat_hbm.at[sid])


def _phase2(comp_hbm, kmat_hbm, cmat_hbm, out_tok, out_val, out_cu,
            comp, fcomp, cidx, pidx, padt, padv, kall, call2, cuv, sem):
  cid = lax.axis_index("c")
  sid = lax.axis_index("s")

  @pl.when(cid == 0)
  def _():
    base = sid * CHUNK
    lane = lax.iota(jnp.int32, 16)

    pltpu.sync_copy(kmat_hbm, kall)
    pltpu.sync_copy(comp_hbm.at[pl.ds(base, CHUNK)], comp)

    def _scan(r, c):
      b, t = c
      kr = kall[r, :][0]
      return (b + jnp.where(r < sid, kr, 0), t + kr)

    base_t, k_tot = lax.fori_loop(0, NT, _scan, (jnp.int32(0), jnp.int32(0)))
    k_t = kall[sid, :][0]

    def _mk(k, _):
      flat = k * 16 + lane
      tok_c = comp[pl.ds(k * 16, 16)]
      fcomp[pl.ds(k * 16, 16)] = tok_c.astype(jnp.float32)
      j = k // 8
      c = k % 8
      cidx[j, pl.ds(c * 16, 16)] = jnp.where(flat < k_t, base_t + flat,
                                             N + flat)
      p = base + flat
      pidx[j, pl.ds(c * 16, 16)] = jnp.where(p >= k_tot, p, N + flat)
      return 0
    lax.fori_loop(0, NV, _mk, 0)

    def _pd(k, _):
      padt[pl.ds(k * 16, 16)] = jnp.full((16,), PAD_TOK, jnp.int32)
      padv[pl.ds(k * 16, 16)] = jnp.zeros((16,), jnp.float32)
      return 0
    lax.fori_loop(0, 8, _pd, 0)

    for j in range(NT):
      cp = pltpu.async_copy(comp.at[pl.ds(128 * j, 128)],
                            out_tok.at[cidx.at[j]], sem)
      cp2 = pltpu.async_copy(fcomp.at[pl.ds(128 * j, 128)],
                             out_val.at[cidx.at[j]], sem)
      cp3 = pltpu.async_copy(padt, out_tok.at[pidx.at[j]], sem)
      cp4 = pltpu.async_copy(padv, out_val.at[pidx.at[j]], sem)
      cp.wait()
      cp2.wait()
      cp3.wait()
      cp4.wait()

    @pl.when(sid == 0)
    def _cu():
      pltpu.sync_copy(cmat_hbm, call2)

      def _sum(r, c):
        acc, run = c
        row = call2[r, :]
        acc = acc + jnp.where(row >= 0, run + row, 0)
        return (acc, run + kall[r, :][0])

      acc, _ = lax.fori_loop(0, NT, _sum,
                             (jnp.zeros((16,), jnp.int32), jnp.int32(0)))
      cuv[pl.ds(0, 16)] = acc
      cuv[pl.ds(16, 16)] = jnp.where(lane == 0, k_tot, 0)
      pltpu.sync_copy(cuv, out_cu)


_p1 = functools.partial(
    pl.kernel,
    out_type=[
        jax.ShapeDtypeStruct((N,), jnp.int32),
        jax.ShapeDtypeStruct((16, 16), jnp.int32),
        jax.ShapeDtypeStruct((16, 16), jnp.int32),
    ],
    mesh=plsc.VectorSubcoreMesh(core_axis_name="c", subcore_axis_name="s"),
    compiler_params=pltpu.CompilerParams(needs_layout_passes=False),
    scratch_types=[
        pltpu.VMEM((LOC,), jnp.int32),
        pltpu.VMEM((LOC,), jnp.int32),
        pltpu.VMEM((CHUNK,), jnp.int32),
        pltpu.VMEM((LOC,), jnp.int32),
        pltpu.VMEM((16,), jnp.int32),
        pltpu.VMEM((16,), jnp.int32),
        pltpu.SemaphoreType.DMA,
    ],
)(_phase1)


_p2 = functools.partial(
    pl.kernel,
    out_type=[
        jax.ShapeDtypeStruct((N + CHUNK,), jnp.int32),
        jax.ShapeDtypeStruct((N + CHUNK,), jnp.float32),
        jax.ShapeDtypeStruct((32,), jnp.int32),
    ],
    mesh=plsc.VectorSubcoreMesh(core_axis_name="c", subcore_axis_name="s"),
    compiler_params=pltpu.CompilerParams(needs_layout_passes=False),
    scratch_types=[
        pltpu.VMEM((CHUNK,), jnp.int32),
        pltpu.VMEM((CHUNK,), jnp.float32),
        pltpu.VMEM((16, 128), jnp.int32),
        pltpu.VMEM((16, 128), jnp.int32),
        pltpu.VMEM((128,), jnp.int32),
        pltpu.VMEM((128,), jnp.float32),
        pltpu.VMEM((16, 16), jnp.int32),
        pltpu.VMEM((16, 16), jnp.int32),
        pltpu.VMEM((32,), jnp.int32),
        pltpu.SemaphoreType.DMA,
    ],
)(_phase2)


def kernel(tokens, cu_seqlens, values_f):
  del values_f
  pad8 = jnp.full((8,), PAD_TOK, jnp.int32)
  tokens_p = jnp.concatenate([pad8, tokens.astype(jnp.int32), pad8])
  bv = jnp.concatenate(
      [jnp.full((1,), BIG, jnp.int32), cu_seqlens[1:16].astype(jnp.int32)])
  comp_s, kmat, cmat = _p1(tokens_p, bv)
  out_tok, out_val, out_cu = _p2(comp_s, kmat, cmat)
  return out_tok[:N], out_val[:N], out_cu[:17]


# speedup vs baseline: 1.3343x; 1.0013x over previous
import functools

import jax
import jax.numpy as jnp
from jax import lax
from jax.experimental import pallas as pl
from jax.experimental.pallas import tpu as pltpu
from jax.experimental.pallas import tpu_sc as plsc

P = 97
OP_ADD = P
OP_SUB = P + 1
OP_MUL = P + 2
PAD_TOK = P + 3 + 3
N = 32768
NT = 16
CHUNK = N // NT
LOC = CHUNK + 16
NV = CHUNK // 16
BIG = 1 << 29


def _phase1(tok_hbm, bv_hbm, comp_hbm, kmat_hbm, cmat_hbm,
            tv, st, pref, comp, bvec, kvec, sem):
  del sem
  cid = lax.axis_index("c")
  sid = lax.axis_index("s")

  @pl.when(cid == 0)
  def _():
    base = sid * CHUNK

    pltpu.sync_copy(tok_hbm.at[pl.ds(base, LOC)], tv)
    pltpu.sync_copy(bv_hbm, bvec)
    bv = bvec[...]

    def _zero(k, _):
      st[pl.ds(k * 16, 16)] = jnp.zeros((16,), jnp.int32)
      return 0
    lax.fori_loop(0, LOC // 16, _zero, 0)
    kb = bv + 8 - base
    bmask = (kb >= 0) & (kb < LOC)
    plsc.store_scatter(st, [jnp.where(bmask, kb, 0)],
                       jnp.ones((16,), jnp.int32), mask=bmask)

    def _step(k, wp):
      l0 = 8 + k * 16
      t0 = tv[pl.ds(l0, 16)]
      tm1 = tv[pl.ds(l0 - 1, 16)]
      tm2 = tv[pl.ds(l0 - 2, 16)]
      tp1 = tv[pl.ds(l0 + 1, 16)]
      tp2 = tv[pl.ds(l0 + 2, 16)]
      s0 = st[pl.ds(l0, 16)] > 0
      sm1 = st[pl.ds(l0 - 1, 16)] > 0
      sp1 = st[pl.ds(l0 + 1, 16)] > 0
      sp2 = st[pl.ds(l0 + 2, 16)] > 0

      def opnd(x):
        return x < P

      def opr(x):
        return (x >= P) & (x < P + 3)

      red0 = opr(t0) & opnd(tm1) & opnd(tm2) & (~s0) & (~sm1)
      red1 = opr(tp1) & opnd(t0) & opnd(tm1) & (~sp1) & (~s0)
      red2 = opr(tp2) & opnd(tp1) & opnd(t0) & (~sp2) & (~sp1)
      keep = ~(red0 | red1)

      r_add = lax.rem(t0 + tp1, P)
      r_sub = lax.rem(t0 - tp1 + P, P)
      r_mul = lax.rem(t0 * tp1, P)
      res = jnp.where(tp2 == OP_ADD, r_add,
                      jnp.where(tp2 == OP_SUB, r_sub, r_mul))
      tok_new = jnp.where(red2, res, t0)

      ki = keep.astype(jnp.int32)
      pref[pl.ds(k * 16, 16)] = wp + plsc.cumsum(ki) - ki
      plsc.store_compressed(comp.at[pl.ds(wp, 16)], tok_new, mask=keep)
      return wp + jnp.sum(ki)

    k_t = lax.fori_loop(0, NV, _step, jnp.int32(0))

    pltpu.sync_copy(comp.at[pl.ds(0, CHUNK)], comp_hbm.at[pl.ds(base, CHUNK)])
    kvec[...] = jnp.full((16,), k_t, jnp.int32)
    pltpu.sync_copy(kvec, kmat_hbm.at[sid])

    loc = bv - base
    inr = (loc >= 0) & (loc < CHUNK)
    g = plsc.load_gather(pref, [jnp.where(inr, loc, 0)])
    kvec[...] = jnp.where(inr, g, -1)
    pltpu.sync_copy(kvec, cmat_hbm.at[sid])


def _phase2(comp_hbm, kmat_hbm, cmat_hbm, out_tok, out_val, out_cu,
            comp, fcomp, cidx, pidx, padt, padv, kall, call2, cuv, sem):
  cid = lax.axis_index("c")
  sid = lax.axis_index("s")

  @pl.when(cid == 0)
  def _():
    base = sid * CHUNK
    lane = lax.iota(jnp.int32, 16)

    pltpu.sync_copy(kmat_hbm, kall)
    pltpu.sync_copy(comp_hbm.at[pl.ds(base, CHUNK)], comp)

    def _scan(r, c):
      b, t = c
      kr = kall[r, :][0]
      return (b + jnp.where(r < sid, kr, 0), t + kr)

    base_t, k_tot = lax.fori_loop(0, NT, _scan, (jnp.int32(0), jnp.int32(0)))
    k_t = kall[sid, :][0]

    def _mk(k, _):
      flat = k * 16 + lane
      tok_c = comp[pl.ds(k * 16, 16)]
      fcomp[pl.ds(k * 16, 16)] = tok_c.astype(jnp.float32)
      j = k // 8
      c = k % 8
      cidx[j, pl.ds(c * 16, 16)] = jnp.where(flat < k_t, base_t + flat,
                                             N + flat)
      p = base + flat
      pidx[j, pl.ds(c * 16, 16)] = jnp.where(p >= k_tot, p, N + flat)
      return 0
    lax.fori_loop(0, NV, _mk, 0)

    def _pd(k, _):
      padt[pl.ds(k * 16, 16)] = jnp.full((16,), PAD_TOK, jnp.int32)
      padv[pl.ds(k * 16, 16)] = jnp.zeros((16,), jnp.float32)
      return 0
    lax.fori_loop(0, 8, _pd, 0)

    cps = []
    for j in range(NT):
      cps.append(pltpu.async_copy(comp.at[pl.ds(128 * j, 128)],
                                  out_tok.at[cidx.at[j]], sem))
      cps.append(pltpu.async_copy(fcomp.at[pl.ds(128 * j, 128)],
                                  out_val.at[cidx.at[j]], sem))
      cps.append(pltpu.async_copy(padt, out_tok.at[pidx.at[j]], sem))
      cps.append(pltpu.async_copy(padv, out_val.at[pidx.at[j]], sem))
    for cp in cps:
      cp.wait()

    @pl.when(sid == 0)
    def _cu():
      pltpu.sync_copy(cmat_hbm, call2)

      def _sum(r, c):
        acc, run = c
        row = call2[r, :]
        acc = acc + jnp.where(row >= 0, run + row, 0)
        return (acc, run + kall[r, :][0])

      acc, _ = lax.fori_loop(0, NT, _sum,
                             (jnp.zeros((16,), jnp.int32), jnp.int32(0)))
      cuv[pl.ds(0, 16)] = acc
      cuv[pl.ds(16, 16)] = jnp.where(lane == 0, k_tot, 0)
      pltpu.sync_copy(cuv, out_cu)


_p1 = functools.partial(
    pl.kernel,
    out_type=[
        jax.ShapeDtypeStruct((N,), jnp.int32),
        jax.ShapeDtypeStruct((16, 16), jnp.int32),
        jax.ShapeDtypeStruct((16, 16), jnp.int32),
    ],
    mesh=plsc.VectorSubcoreMesh(core_axis_name="c", subcore_axis_name="s"),
    compiler_params=pltpu.CompilerParams(needs_layout_passes=False),
    scratch_types=[
        pltpu.VMEM((LOC,), jnp.int32),
        pltpu.VMEM((LOC,), jnp.int32),
        pltpu.VMEM((CHUNK,), jnp.int32),
        pltpu.VMEM((LOC,), jnp.int32),
        pltpu.VMEM((16,), jnp.int32),
        pltpu.VMEM((16,), jnp.int32),
        pltpu.SemaphoreType.DMA,
    ],
)(_phase1)


_p2 = functools.partial(
    pl.kernel,
    out_type=[
        jax.ShapeDtypeStruct((N + CHUNK,), jnp.int32),
        jax.ShapeDtypeStruct((N + CHUNK,), jnp.float32),
        jax.ShapeDtypeStruct((32,), jnp.int32),
    ],
    mesh=plsc.VectorSubcoreMesh(core_axis_name="c", subcore_axis_name="s"),
    compiler_params=pltpu.CompilerParams(needs_layout_passes=False),
    scratch_types=[
        pltpu.VMEM((CHUNK,), jnp.int32),
        pltpu.VMEM((CHUNK,), jnp.float32),
        pltpu.VMEM((16, 128), jnp.int32),
        pltpu.VMEM((16, 128), jnp.int32),
        pltpu.VMEM((128,), jnp.int32),
        pltpu.VMEM((128,), jnp.float32),
        pltpu.VMEM((16, 16), jnp.int32),
        pltpu.VMEM((16, 16), jnp.int32),
        pltpu.VMEM((32,), jnp.int32),
        pltpu.SemaphoreType.DMA,
    ],
)(_phase2)


def kernel(tokens, cu_seqlens, values_f):
  del values_f
  pad8 = jnp.full((8,), PAD_TOK, jnp.int32)
  tokens_p = jnp.concatenate([pad8, tokens.astype(jnp.int32), pad8])
  bv = jnp.concatenate(
      [jnp.full((1,), BIG, jnp.int32), cu_seqlens[1:16].astype(jnp.int32)])
  comp_s, kmat, cmat = _p1(tokens_p, bv)
  out_tok, out_val, out_cu = _p2(comp_s, kmat, cmat)
  return out_tok[:N], out_val[:N], out_cu[:17]


# speedup vs baseline: 5.8219x; 4.3635x over previous
import functools

import jax
import jax.numpy as jnp
from jax import lax
from jax.experimental import pallas as pl
from jax.experimental.pallas import tpu as pltpu
from jax.experimental.pallas import tpu_sc as plsc

P = 97
OP_ADD = P
OP_SUB = P + 1
OP_MUL = P + 2
PAD_TOK = P + 3 + 3
N = 32768
NT = 16
CHUNK = N // NT
LOC = CHUNK + 16
NV = CHUNK // 16
BIG = 1 << 29


def _phase1(tok_hbm, bv_hbm, comp_hbm, kmat_hbm, cmat_hbm,
            tv, st, pref, comp, bvec, kvec, sem):
  del sem
  cid = lax.axis_index("c")
  sid = lax.axis_index("s")

  @pl.when(cid == 0)
  def _():
    base = sid * CHUNK

    pltpu.sync_copy(tok_hbm.at[pl.ds(base, LOC)], tv)
    pltpu.sync_copy(bv_hbm, bvec)
    bv = bvec[...]

    def _zero(k, _):
      st[pl.ds(k * 16, 16)] = jnp.zeros((16,), jnp.int32)
      return 0
    lax.fori_loop(0, LOC // 16, _zero, 0)
    kb = bv + 8 - base
    bmask = (kb >= 0) & (kb < LOC)
    plsc.store_scatter(st, [jnp.where(bmask, kb, 0)],
                       jnp.ones((16,), jnp.int32), mask=bmask)

    def _step(k, wp):
      l0 = 8 + k * 16
      t0 = tv[pl.ds(l0, 16)]
      tm1 = tv[pl.ds(l0 - 1, 16)]
      tm2 = tv[pl.ds(l0 - 2, 16)]
      tp1 = tv[pl.ds(l0 + 1, 16)]
      tp2 = tv[pl.ds(l0 + 2, 16)]
      s0 = st[pl.ds(l0, 16)] > 0
      sm1 = st[pl.ds(l0 - 1, 16)] > 0
      sp1 = st[pl.ds(l0 + 1, 16)] > 0
      sp2 = st[pl.ds(l0 + 2, 16)] > 0

      def opnd(x):
        return x < P

      def opr(x):
        return (x >= P) & (x < P + 3)

      red0 = opr(t0) & opnd(tm1) & opnd(tm2) & (~s0) & (~sm1)
      red1 = opr(tp1) & opnd(t0) & opnd(tm1) & (~sp1) & (~s0)
      red2 = opr(tp2) & opnd(tp1) & opnd(t0) & (~sp2) & (~sp1)
      keep = ~(red0 | red1)

      r_add = lax.rem(t0 + tp1, P)
      r_sub = lax.rem(t0 - tp1 + P, P)
      r_mul = lax.rem(t0 * tp1, P)
      res = jnp.where(tp2 == OP_ADD, r_add,
                      jnp.where(tp2 == OP_SUB, r_sub, r_mul))
      tok_new = jnp.where(red2, res, t0)

      ki = keep.astype(jnp.int32)
      pref[pl.ds(k * 16, 16)] = wp + plsc.cumsum(ki) - ki
      plsc.store_compressed(comp.at[pl.ds(wp, 16)], tok_new, mask=keep)
      return wp + jnp.sum(ki)

    k_t = lax.fori_loop(0, NV, _step, jnp.int32(0))

    pltpu.sync_copy(comp.at[pl.ds(0, CHUNK)], comp_hbm.at[pl.ds(base, CHUNK)])
    kvec[...] = jnp.full((16,), k_t, jnp.int32)
    pltpu.sync_copy(kvec, kmat_hbm.at[sid])

    loc = bv - base
    inr = (loc >= 0) & (loc < CHUNK)
    g = plsc.load_gather(pref, [jnp.where(inr, loc, 0)])
    kvec[...] = jnp.where(inr, g, -1)
    pltpu.sync_copy(kvec, cmat_hbm.at[sid])


def _phase2(comp_hbm, kmat_hbm, cmat_hbm, out_tok, out_val, out_cu,
            comp, fcomp, cidx, pidx, padt, padv,
            kall, call2, cuv, sem):
  cid = lax.axis_index("c")
  sid = lax.axis_index("s")

  @pl.when(cid == 0)
  def _():
    base = sid * CHUNK
    lane = lax.iota(jnp.int32, 16)

    pltpu.sync_copy(kmat_hbm, kall)
    pltpu.sync_copy(comp_hbm.at[pl.ds(base, CHUNK)], comp)

    def _scan(r, c):
      b, t = c
      kr = kall[r, :][0]
      return (b + jnp.where(r < sid, kr, 0), t + kr)

    base_t, k_tot = lax.fori_loop(0, NT, _scan, (jnp.int32(0), jnp.int32(0)))
    k_t = kall[sid, :][0]

    def _mk(k, _):
      flat = k * 16 + lane
      tok_c = comp[pl.ds(k * 16, 16)]
      fcomp[pl.ds(k * 16, 16)] = tok_c.astype(jnp.float32)
      cidx[pl.ds(k * 16, 16)] = jnp.where(flat < k_t, base_t + flat,
                                          N + base + flat)
      p = base + flat
      pidx[pl.ds(k * 16, 16)] = jnp.where(p >= k_tot, p, N + base + flat)
      padt[pl.ds(k * 16, 16)] = jnp.full((16,), PAD_TOK, jnp.int32)
      padv[pl.ds(k * 16, 16)] = jnp.zeros((16,), jnp.float32)
      return 0
    lax.fori_loop(0, NV, _mk, 0)

    c1 = pltpu.async_copy(comp.at[pl.ds(0, CHUNK)], out_tok.at[cidx], sem)
    c2 = pltpu.async_copy(fcomp, out_val.at[cidx], sem)
    c1.wait()
    c2.wait()

    @pl.when(base + CHUNK > k_tot)
    def _padfill():
      p1 = pltpu.async_copy(padt, out_tok.at[pidx], sem)
      p2 = pltpu.async_copy(padv, out_val.at[pidx], sem)
      p1.wait()
      p2.wait()

    @pl.when(sid == 0)
    def _cu():
      pltpu.sync_copy(cmat_hbm, call2)

      def _sum(r, c):
        acc, run = c
        row = call2[r, :]
        acc = acc + jnp.where(row >= 0, run + row, 0)
        return (acc, run + kall[r, :][0])

      acc, _ = lax.fori_loop(0, NT, _sum,
                             (jnp.zeros((16,), jnp.int32), jnp.int32(0)))
      cuv[pl.ds(0, 16)] = acc
      cuv[pl.ds(16, 16)] = jnp.where(lane == 0, k_tot, 0)
      pltpu.sync_copy(cuv, out_cu)


_p1 = functools.partial(
    pl.kernel,
    out_type=[
        jax.ShapeDtypeStruct((N,), jnp.int32),
        jax.ShapeDtypeStruct((16, 16), jnp.int32),
        jax.ShapeDtypeStruct((16, 16), jnp.int32),
    ],
    mesh=plsc.VectorSubcoreMesh(core_axis_name="c", subcore_axis_name="s"),
    compiler_params=pltpu.CompilerParams(needs_layout_passes=False),
    scratch_types=[
        pltpu.VMEM((LOC,), jnp.int32),
        pltpu.VMEM((LOC,), jnp.int32),
        pltpu.VMEM((CHUNK,), jnp.int32),
        pltpu.VMEM((LOC,), jnp.int32),
        pltpu.VMEM((16,), jnp.int32),
        pltpu.VMEM((16,), jnp.int32),
        pltpu.SemaphoreType.DMA,
    ],
)(_phase1)


_p2 = functools.partial(
    pl.kernel,
    out_type=[
        jax.ShapeDtypeStruct((2 * N,), jnp.int32),
        jax.ShapeDtypeStruct((2 * N,), jnp.float32),
        jax.ShapeDtypeStruct((32,), jnp.int32),
    ],
    mesh=plsc.VectorSubcoreMesh(core_axis_name="c", subcore_axis_name="s"),
    compiler_params=pltpu.CompilerParams(needs_layout_passes=False),
    scratch_types=[
        pltpu.VMEM((CHUNK,), jnp.int32),
        pltpu.VMEM((CHUNK,), jnp.float32),
        pltpu.VMEM((CHUNK,), jnp.int32),
        pltpu.VMEM((CHUNK,), jnp.int32),
        pltpu.VMEM((CHUNK,), jnp.int32),
        pltpu.VMEM((CHUNK,), jnp.float32),
        pltpu.VMEM((16, 16), jnp.int32),
        pltpu.VMEM((16, 16), jnp.int32),
        pltpu.VMEM((32,), jnp.int32),
        pltpu.SemaphoreType.DMA,
    ],
)(_phase2)


def kernel(tokens, cu_seqlens, values_f):
  del values_f
  pad8 = jnp.full((8,), PAD_TOK, jnp.int32)
  tokens_p = jnp.concatenate([pad8, tokens.astype(jnp.int32), pad8])
  bv = jnp.concatenate(
      [jnp.full((1,), BIG, jnp.int32), cu_seqlens[1:16].astype(jnp.int32)])
  comp_s, kmat, cmat = _p1(tokens_p, bv)
  out_tok, out_val, out_cu = _p2(comp_s, kmat, cmat)
  return out_tok[:N], out_val[:N], out_cu[:17]
